# baseline (speedup 1.0000x reference)
import functools

import jax
import jax.numpy as jnp
from jax import lax
from jax.experimental import pallas as pl
from jax.experimental.pallas import tpu as pltpu
from jax.experimental.pallas import tpu_sc as plsc

BATCH = 4096
SEQ = 200
D_MODEL = 64
VOCAB = 1000000
NUM_CORES = 2
NUM_SUBCORES = 16
N_XTILES = (SEQ // 8) * (BATCH // 128)
TILES_PER_TEC = N_XTILES // NUM_SUBCORES
D_PER_CORE = D_MODEL // NUM_CORES
ROW_CHUNK = 62464
LAST_START = 15 * ROW_CHUNK
LAST_CHUNK = VOCAB - LAST_START
N_DST = 4


def _sc_embed_permute(xq, tq):
    mesh = plsc.VectorSubcoreMesh(core_axis_name="c", subcore_axis_name="s")

    @functools.partial(
        pl.kernel,
        mesh=mesh,
        compiler_params=pltpu.CompilerParams(
            needs_layout_passes=False, use_tc_tiling_on_sc=False
        ),
        out_type=jax.ShapeDtypeStruct((D_MODEL, N_XTILES, 1024), jnp.float32),
        scratch_types=[
            pltpu.VMEM((TILES_PER_TEC, 1024), jnp.int32),
            pltpu.VMEM((N_DST, 1024), jnp.float32),
            pltpu.VMEM_SHARED((VOCAB,), jnp.float32),
            pltpu.SemaphoreType.DMA,
            pltpu.SemaphoreType.DMA,
            pltpu.SemaphoreType.DMA,
        ],
    )
    def k(xq_hbm, tq_hbm, o_hbm, idx_cache, dst, sp, rs, sg, sw):
        cid = lax.axis_index("c")
        tid = lax.axis_index("s")
        d_base = cid * D_PER_CORE

        def row_load(dg):
            @pl.when(tid < 15)
            def _():
                off = tid * ROW_CHUNK
                pltpu.async_copy(
                    tq_hbm.at[dg, pl.ds(off, ROW_CHUNK)],
                    sp.at[pl.ds(off, ROW_CHUNK)],
                    rs,
                )

            @pl.when(tid == 15)
            def _():
                pltpu.async_copy(
                    tq_hbm.at[dg, pl.ds(LAST_START, LAST_CHUNK)],
                    sp.at[pl.ds(LAST_START, LAST_CHUNK)],
                    rs,
                )

        def row_wait():
            @pl.when(tid < 15)
            def _():
                pltpu.make_async_copy(
                    tq_hbm.at[0, pl.ds(0, ROW_CHUNK)],
                    sp.at[pl.ds(0, ROW_CHUNK)],
                    rs,
                ).wait()

            @pl.when(tid == 15)
            def _():
                pltpu.make_async_copy(
                    tq_hbm.at[0, pl.ds(0, LAST_CHUNK)],
                    sp.at[pl.ds(0, LAST_CHUNK)],
                    rs,
                ).wait()

        def wait_write():
            pltpu.make_async_copy(dst.at[0], o_hbm.at[0, 0], sw).wait()

        def wait_gather():
            pltpu.make_async_copy(sp.at[idx_cache.at[0]], dst.at[0], sg).wait()

        def units(dg):
            for u in range(TILES_PER_TEC):
                b = u % N_DST
                if u >= N_DST:
                    wait_write()
                pltpu.async_copy(sp.at[idx_cache.at[u]], dst.at[b], sg)
                if u > 0:
                    wait_gather()
                    bp = (u - 1) % N_DST
                    pltpu.async_copy(
                        dst.at[bp], o_hbm.at[dg, tid * TILES_PER_TEC + (u - 1)], sw
                    )
            wait_gather()
            last = TILES_PER_TEC - 1
            pltpu.async_copy(
                dst.at[last % N_DST],
                o_hbm.at[dg, tid * TILES_PER_TEC + last],
                sw,
            )
            for _ in range(N_DST):
                wait_write()

        pltpu.sync_copy(xq_hbm.at[pl.ds(tid * TILES_PER_TEC, TILES_PER_TEC)],
                        idx_cache)

        def body(dd, carry):
            row_load(d_base + dd)
            row_wait()
            plsc.subcore_barrier()
            units(d_base + dd)
            plsc.subcore_barrier()
            return carry

        lax.fori_loop(0, D_PER_CORE, body, 0)

    return k(xq, tq)


def kernel(x, table):
    xq = (
        x.astype(jnp.int32)
        .T.reshape(25, 8, 32, 128)
        .transpose(0, 2, 1, 3)
        .reshape(N_XTILES, 1024)
    )
    tq = table.T
    o5 = _sc_embed_permute(xq, tq)
    return (
        o5.reshape(64, 25, 32, 8, 128)
        .transpose(2, 4, 0, 1, 3)
        .reshape(BATCH, D_MODEL, SEQ)
    )


# speedup vs baseline: 3.4651x; 3.4651x over previous
import functools

import jax
import jax.numpy as jnp
from jax import lax
from jax.experimental import pallas as pl
from jax.experimental.pallas import tpu as pltpu
from jax.experimental.pallas import tpu_sc as plsc

BATCH = 4096
SEQ = 200
D_MODEL = 64
VOCAB = 1000000
NUM_CORES = 2
NUM_SUBCORES = 16
N_TECS = NUM_CORES * NUM_SUBCORES
N_XTILES = (SEQ // 8) * (BATCH // 128)
UNITS_PER_TEC = N_XTILES // N_TECS
QJ = 256
N_Q = 1024 // QJ


def _sc_embed_permute(xq, tq):
    mesh = plsc.VectorSubcoreMesh(core_axis_name="c", subcore_axis_name="s")

    @functools.partial(
        pl.kernel,
        mesh=mesh,
        compiler_params=pltpu.CompilerParams(
            needs_layout_passes=False, use_tc_tiling_on_sc=False
        ),
        out_type=jax.ShapeDtypeStruct((D_MODEL, N_XTILES, 1024), jnp.float32),
        scratch_types=[
            pltpu.VMEM((UNITS_PER_TEC, 1024), jnp.int32),
            pltpu.VMEM((QJ, D_MODEL), jnp.float32),
            pltpu.VMEM((QJ, D_MODEL), jnp.float32),
            pltpu.VMEM((D_MODEL, 1024), jnp.float32),
            pltpu.SemaphoreType.DMA,
            pltpu.SemaphoreType.DMA,
        ],
    )
    def k(xq_hbm, tq_hbm, o_hbm, idx_cache, rows0, rows1, outs, sg, sw):
        cid = lax.axis_index("c")
        tid = lax.axis_index("s")
        w = cid * NUM_SUBCORES + tid
        t_base = w * UNITS_PER_TEC

        rows = (rows0, rows1)
        iota = lax.iota(jnp.int32, 16)
        dvecs = [iota + dc * 16 for dc in range(D_MODEL // 16)]

        def start_gather(u, q, buf):
            pltpu.async_copy(
                tq_hbm.at[idx_cache.at[u, pl.ds(q * QJ, QJ)]], buf, sg
            )

        def wait_gather():
            pltpu.make_async_copy(
                tq_hbm.at[pl.ds(0, QJ)], rows0, sg
            ).wait()

        def wait_write():
            pltpu.make_async_copy(outs, o_hbm.at[:, 0], sw).wait()

        def transpose(buf, q):
            def jbody(j4, carry):
                for v in range(4):
                    j = j4 * 4 + v
                    col = jnp.full((16,), 0, jnp.int32) + (q * QJ + j)
                    for dc in range(D_MODEL // 16):
                        vals = buf[j, pl.ds(dc * 16, 16)]
                        plsc.store_scatter(outs, [dvecs[dc], col], vals)
                return carry

            lax.fori_loop(0, QJ // 4, jbody, 0)

        pltpu.sync_copy(xq_hbm.at[pl.ds(t_base, UNITS_PER_TEC)], idx_cache)
        start_gather(0, 0, rows[0])

        def body(u, carry):
            @pl.when(u > 0)
            def _():
                wait_write()

            for q in range(N_Q):
                p = q % 2
                if q < N_Q - 1:
                    start_gather(u, q + 1, rows[1 - p])
                else:
                    un = jnp.minimum(u + 1, UNITS_PER_TEC - 1)
                    start_gather(un, 0, rows[1 - p])
                wait_gather()
                transpose(rows[p], q)

            pltpu.async_copy(outs, o_hbm.at[:, t_base + u], sw)
            return carry

        lax.fori_loop(0, UNITS_PER_TEC, body, 0)
        wait_gather()
        wait_write()

    return k(xq, tq)


def kernel(x, table):
    xq = (
        x.astype(jnp.int32)
        .T.reshape(25, 8, 32, 128)
        .transpose(0, 2, 1, 3)
        .reshape(N_XTILES, 1024)
    )
    o5 = _sc_embed_permute(xq, table)
    return (
        o5.reshape(64, 25, 32, 8, 128)
        .transpose(2, 4, 0, 1, 3)
        .reshape(BATCH, D_MODEL, SEQ)
    )
